# baseline (speedup 1.0000x reference)
import jax
import jax.numpy as jnp
from jax import lax
from jax.experimental import pallas as pl
from jax.experimental.pallas import tpu as pltpu
from jax.experimental.pallas import tpu_sc as plsc

_SIZES = (500, 200, 2000, 1000, 300)
_N_IN = 10000
_ROWS = 4 * 2048

_NC, _NS, _L = 2, 16, 16
_NW = _NC * _NS

_SIZES_PAD = tuple(-(-s // _L) * _L for s in _SIZES)
_OFF_PAD = (0, 512, 720, 2720, 3728)
_TOTAL_PAD = sum(_SIZES_PAD)

_ROWS_PER_W = _ROWS // _NW
_RB = 4
_NBLK = _ROWS_PER_W // _RB


def _gather_segment(xbuf, obuf_k, ids_v, off_pad, size):
    nfull = size // _L
    tail = size % _L

    def chunk(c, carry):
        idx = ids_v[pl.ds(off_pad + c * _L, _L)]
        for r in range(_RB):
            vals = plsc.load_gather(xbuf, [idx + (r * _N_IN)])
            obuf_k[r, pl.ds(c * _L, _L)] = vals
        return carry

    lax.fori_loop(0, nfull, chunk, 0)

    if tail:
        lane = lax.iota(jnp.int32, _L)
        mask = lane < tail
        pos = jnp.full((_L,), nfull * _L, jnp.int32) + lane
        idx = ids_v[pl.ds(off_pad + nfull * _L, _L)]
        for r in range(_RB):
            vals = plsc.load_gather(xbuf, [idx + (r * _N_IN)])
            plsc.store_scatter(
                obuf_k, [jnp.full((_L,), r, jnp.int32), pos], vals, mask=mask
            )


def _body(x_ref, ids_ref, o0, o1, o2, o3, o4,
          ids_v, xbuf, ob0, ob1, ob2, ob3, ob4, sem_in, sem_out):
    outs = (o0, o1, o2, o3, o4)
    obufs = (ob0, ob1, ob2, ob3, ob4)
    wid = lax.axis_index("s") * _NC + lax.axis_index("c")
    base_row = wid * _ROWS_PER_W

    pltpu.sync_copy(ids_ref, ids_v)

    def blk(b, carry):
        row0 = base_row + b * _RB
        pltpu.async_copy(
            x_ref.at[pl.ds(row0 * _N_IN, _RB * _N_IN)], xbuf, sem_in
        ).wait()

        for k in range(5):
            _gather_segment(xbuf, obufs[k], ids_v, _OFF_PAD[k], _SIZES[k])

        for k in range(5):
            pltpu.async_copy(
                obufs[k], outs[k].at[pl.ds(row0, _RB)], sem_out
            )
        for k in range(5):
            pltpu.make_async_copy(
                obufs[k], outs[k].at[pl.ds(row0, _RB)], sem_out
            ).wait()
        return carry

    lax.fori_loop(0, _NBLK, blk, 0)


@jax.jit
def kernel(x, cat_ids):
    b, t, n = x.shape
    x_flat = x.reshape(b * t * n)

    segs = []
    off = 0
    for s, sp in zip(_SIZES, _SIZES_PAD):
        seg = lax.dynamic_slice(cat_ids, (off,), (s,))
        segs.append(jnp.pad(seg, (0, sp - s)))
        off += s
    ids_pad = jnp.concatenate(segs)

    mesh = plsc.VectorSubcoreMesh(
        core_axis_name="c", subcore_axis_name="s", num_cores=_NC, num_subcores=_NS
    )
    out_type = tuple(
        jax.ShapeDtypeStruct((_ROWS, s), jnp.float32) for s in _SIZES
    )
    fn = pl.kernel(
        _body,
        out_type=out_type,
        mesh=mesh,
        compiler_params=pltpu.CompilerParams(needs_layout_passes=False),
        scratch_types=[
            pltpu.VMEM((_TOTAL_PAD,), jnp.int32),
            pltpu.VMEM((_RB * _N_IN,), jnp.float32),
        ] + [
            pltpu.VMEM((_RB, s), jnp.float32) for s in _SIZES
        ] + [
            pltpu.SemaphoreType.DMA,
            pltpu.SemaphoreType.DMA,
        ],
    )
    outs = fn(x_flat, ids_pad)
    return tuple(o.reshape(b, t, s) for o, s in zip(outs, _SIZES))


# speedup vs baseline: 1.2046x; 1.2046x over previous
import jax
import jax.numpy as jnp
from jax import lax
from jax.experimental import pallas as pl
from jax.experimental.pallas import tpu as pltpu
from jax.experimental.pallas import tpu_sc as plsc

_SIZES = (500, 200, 2000, 1000, 300)
_N_IN = 10000
_ROWS = 4 * 2048

_NC, _NS, _L = 2, 16, 16
_NW = _NC * _NS

_SIZES_PAD = tuple(-(-s // _L) * _L for s in _SIZES)
_OFF_PAD = (0, 512, 720, 2720, 3728)
_TOTAL_PAD = sum(_SIZES_PAD)

_ROWS_PER_W = _ROWS // _NW
_RB = 4
_NBLK = _ROWS_PER_W // _RB


def _gather_segment(xbuf, obuf_k, ids_v, off_pad, size):
    nfull = size // _L
    tail = size % _L

    def chunk(c, carry):
        idx = ids_v[pl.ds(off_pad + c * _L, _L)]
        for r in range(_RB):
            vals = plsc.load_gather(xbuf, [idx + (r * _N_IN)])
            obuf_k[r, pl.ds(c * _L, _L)] = vals
        return carry

    lax.fori_loop(0, nfull, chunk, 0, unroll=4)

    if tail:
        lane = lax.iota(jnp.int32, _L)
        mask = lane < tail
        pos = jnp.full((_L,), nfull * _L, jnp.int32) + lane
        idx = ids_v[pl.ds(off_pad + nfull * _L, _L)]
        for r in range(_RB):
            vals = plsc.load_gather(xbuf, [idx + (r * _N_IN)])
            plsc.store_scatter(
                obuf_k, [jnp.full((_L,), r, jnp.int32), pos], vals, mask=mask
            )


def _body(x_ref, ids_ref, o0, o1, o2, o3, o4,
          ids_v, xa, xb,
          oa0, oa1, oa2, oa3, oa4, ob0, ob1, ob2, ob3, ob4,
          sem_in_a, sem_in_b, sem_out_a, sem_out_b):
    outs = (o0, o1, o2, o3, o4)
    obufs = ((oa0, oa1, oa2, oa3, oa4), (ob0, ob1, ob2, ob3, ob4))
    xbufs = (xa, xb)
    sems_in = (sem_in_a, sem_in_b)
    sems_out = (sem_out_a, sem_out_b)

    wid = lax.axis_index("s") * _NC + lax.axis_index("c")
    base_row = wid * _ROWS_PER_W

    pltpu.sync_copy(ids_ref, ids_v)

    def in_copy(b, slot):
        return pltpu.make_async_copy(
            x_ref.at[pl.ds((base_row + b * _RB) * _N_IN, _RB * _N_IN)],
            xbufs[slot], sems_in[slot],
        )

    def out_copies(b, slot):
        row0 = base_row + b * _RB
        return [
            pltpu.make_async_copy(
                obufs[slot][k], outs[k].at[pl.ds(row0, _RB)], sems_out[slot]
            )
            for k in range(5)
        ]

    in_copy(0, 0).start()

    def pair(g, carry):
        for slot in range(2):
            b = 2 * g + slot
            nxt = b + 1

            @pl.when(nxt < _NBLK)
            def _():
                in_copy(nxt, 1 - slot).start()

            in_copy(b, slot).wait()

            @pl.when(g > 0)
            def _():
                for c in out_copies(b, slot):
                    c.wait()

            for k in range(5):
                _gather_segment(xbufs[slot], obufs[slot][k], ids_v,
                                _OFF_PAD[k], _SIZES[k])
            for c in out_copies(b, slot):
                c.start()
        return carry

    lax.fori_loop(0, _NBLK // 2, pair, 0)

    for slot in range(2):
        for c in out_copies(_NBLK - 2 + slot, slot):
            c.wait()


@jax.jit
def kernel(x, cat_ids):
    b, t, n = x.shape
    x_flat = x.reshape(b * t * n)

    segs = []
    off = 0
    for s, sp in zip(_SIZES, _SIZES_PAD):
        seg = lax.dynamic_slice(cat_ids, (off,), (s,))
        segs.append(jnp.pad(seg, (0, sp - s)))
        off += s
    ids_pad = jnp.concatenate(segs)

    mesh = plsc.VectorSubcoreMesh(
        core_axis_name="c", subcore_axis_name="s", num_cores=_NC, num_subcores=_NS
    )
    out_type = tuple(
        jax.ShapeDtypeStruct((_ROWS, s), jnp.float32) for s in _SIZES
    )
    fn = pl.kernel(
        _body,
        out_type=out_type,
        mesh=mesh,
        compiler_params=pltpu.CompilerParams(needs_layout_passes=False),
        scratch_types=[
            pltpu.VMEM((_TOTAL_PAD,), jnp.int32),
            pltpu.VMEM((_RB * _N_IN,), jnp.float32),
            pltpu.VMEM((_RB * _N_IN,), jnp.float32),
        ] + [
            pltpu.VMEM((_RB, s), jnp.float32) for s in _SIZES
        ] + [
            pltpu.VMEM((_RB, s), jnp.float32) for s in _SIZES
        ] + [
            pltpu.SemaphoreType.DMA,
            pltpu.SemaphoreType.DMA,
            pltpu.SemaphoreType.DMA,
            pltpu.SemaphoreType.DMA,
        ],
    )
    outs = fn(x_flat, ids_pad)
    return tuple(o.reshape(b, t, s) for o, s in zip(outs, _SIZES))


# speedup vs baseline: 1.4179x; 1.1771x over previous
import jax
import jax.numpy as jnp
from jax import lax
from jax.experimental import pallas as pl
from jax.experimental.pallas import tpu as pltpu
from jax.experimental.pallas import tpu_sc as plsc

_SIZES = (500, 200, 2000, 1000, 300)
_N_IN = 10000
_ROWS = 4 * 2048

_NC, _NS, _L = 2, 16, 16
_NW = _NC * _NS

_SIZES_PAD = tuple(-(-s // _L) * _L for s in _SIZES)
_OFF_PAD = (0, 512, 720, 2720, 3728)
_TOTAL_PAD = sum(_SIZES_PAD)

_ROWS_PER_W = _ROWS // _NW
_RB = 8
_NBLK = _ROWS_PER_W // _RB


def _gather_segment(xbuf, obuf_k, ids_v, row_splats, off_pad, size):
    nfull = size // _L
    tail = size % _L

    def chunk(c, carry):
        idx = ids_v[pl.ds(off_pad + c * _L, _L)]
        for r in range(_RB):
            vals = plsc.load_gather(xbuf, [row_splats[r], idx])
            obuf_k[r, pl.ds(c * _L, _L)] = vals
        return carry

    lax.fori_loop(0, nfull, chunk, 0, unroll=4)

    if tail:
        lane = lax.iota(jnp.int32, _L)
        mask = lane < tail
        pos = jnp.full((_L,), nfull * _L, jnp.int32) + lane
        idx = ids_v[pl.ds(off_pad + nfull * _L, _L)]
        for r in range(_RB):
            vals = plsc.load_gather(xbuf, [row_splats[r], idx])
            plsc.store_scatter(
                obuf_k, [row_splats[r], pos], vals, mask=mask
            )


def _body(x_ref, ids_ref, o0, o1, o2, o3, o4,
          ids_v, xbuf, ob0, ob1, ob2, ob3, ob4, sem_in, sem_out):
    outs = (o0, o1, o2, o3, o4)
    obufs = (ob0, ob1, ob2, ob3, ob4)
    wid = lax.axis_index("s") * _NC + lax.axis_index("c")
    base_row = wid * _ROWS_PER_W

    pltpu.sync_copy(ids_ref, ids_v)

    row_splats = [jnp.full((_L,), r, jnp.int32) for r in range(_RB)]

    def blk(b, carry):
        row0 = base_row + b * _RB
        pltpu.async_copy(
            x_ref.at[pl.ds(row0, _RB)], xbuf, sem_in
        ).wait()

        for k in range(5):
            _gather_segment(xbuf, obufs[k], ids_v, row_splats,
                            _OFF_PAD[k], _SIZES[k])

        for k in range(5):
            pltpu.async_copy(
                obufs[k], outs[k].at[pl.ds(row0, _RB)], sem_out
            )
        for k in range(5):
            pltpu.make_async_copy(
                obufs[k], outs[k].at[pl.ds(row0, _RB)], sem_out
            ).wait()
        return carry

    lax.fori_loop(0, _NBLK, blk, 0)


@jax.jit
def kernel(x, cat_ids):
    b, t, n = x.shape
    x2d = x.reshape(b * t, n)

    segs = []
    off = 0
    for s, sp in zip(_SIZES, _SIZES_PAD):
        seg = lax.dynamic_slice(cat_ids, (off,), (s,))
        segs.append(jnp.pad(seg, (0, sp - s)))
        off += s
    ids_pad = jnp.concatenate(segs)

    mesh = plsc.VectorSubcoreMesh(
        core_axis_name="c", subcore_axis_name="s", num_cores=_NC, num_subcores=_NS
    )
    out_type = tuple(
        jax.ShapeDtypeStruct((_ROWS, s), jnp.float32) for s in _SIZES
    )
    fn = pl.kernel(
        _body,
        out_type=out_type,
        mesh=mesh,
        compiler_params=pltpu.CompilerParams(needs_layout_passes=False),
        scratch_types=[
            pltpu.VMEM((_TOTAL_PAD,), jnp.int32),
            pltpu.VMEM((_RB, _N_IN), jnp.float32),
        ] + [
            pltpu.VMEM((_RB, s), jnp.float32) for s in _SIZES
        ] + [
            pltpu.SemaphoreType.DMA,
            pltpu.SemaphoreType.DMA,
        ],
    )
    outs = fn(x2d, ids_pad)
    return tuple(o.reshape(b, t, s) for o, s in zip(outs, _SIZES))
